# baseline (speedup 1.0000x reference)
import functools

import jax
import jax.numpy as jnp
from jax.experimental import pallas as pl
from jax.experimental.pallas import tpu as pltpu

_F32 = jnp.float32
_VMEM_LIMIT = 96 * 1024 * 1024


def _tile_spec(block_shape):
    nd = len(block_shape)
    return pl.BlockSpec(block_shape, lambda i, _nd=nd: (i,) + (0,) * (_nd - 1))


def _const_spec(shape):
    nd = len(shape)
    return pl.BlockSpec(shape, lambda i, _nd=nd: (0,) * _nd)


def _edge_mlp_body(ea_ref, k1w_ref, k1b_ref, k2w_ref, k2b_ref, o_ref):
    e = jnp.dot(ea_ref[...], k1w_ref[...], preferred_element_type=_F32) + k1b_ref[...]
    e = jnp.maximum(e, 0.0)
    e = jnp.dot(e, k2w_ref[...], preferred_element_type=_F32) + k2b_ref[...]
    o_ref[...] = jnp.maximum(e, 0.0)


def _edge_mlp(ea, k1w, k1b, k2w, k2b, *, tile):
    e_pad, k_pad = ea.shape
    wk = k2w.shape[1]
    return pl.pallas_call(
        _edge_mlp_body,
        out_shape=jax.ShapeDtypeStruct((e_pad, wk), _F32),
        grid=(e_pad // tile,),
        in_specs=[_tile_spec((tile, k_pad)),
                  _const_spec(k1w.shape), _const_spec(k1b.shape),
                  _const_spec(k2w.shape), _const_spec(k2b.shape)],
        out_specs=_tile_spec((tile, wk)),
        compiler_params=pltpu.CompilerParams(
            dimension_semantics=("parallel",),
            vmem_limit_bytes=_VMEM_LIMIT,
        ),
    )(ea, k1w, k1b, k2w, k2b)


def _msg_body(r2_ref, xs_ref, sc_ref, k3w_ref, k3b_ref, o_ref):
    wflat = jnp.dot(r2_ref[...], k3w_ref[...], preferred_element_type=_F32)
    wflat = wflat + k3b_ref[...]
    xr = jnp.repeat(xs_ref[...], 32, axis=1)
    prod = xr * wflat
    s = prod[:, 0:128]
    for k in range(1, 8):
        s = s + prod[:, 128 * k:128 * (k + 1)]
    msg = (s[:, 0:32] + s[:, 32:64]) + (s[:, 64:96] + s[:, 96:128])
    o_ref[...] = msg * sc_ref[...]


def _messages(relu2, xs, scale, k3w, k3b, *, tile):
    e_pad = relu2.shape[0]
    width = xs.shape[1]
    return pl.pallas_call(
        _msg_body,
        out_shape=jax.ShapeDtypeStruct((e_pad, width), _F32),
        grid=(e_pad // tile,),
        in_specs=[_tile_spec((tile, relu2.shape[1])),
                  _tile_spec((tile, width)),
                  _tile_spec((tile, 1)),
                  _const_spec(k3w.shape), _const_spec(k3b.shape)],
        out_specs=_tile_spec((tile, width)),
        compiler_params=pltpu.CompilerParams(
            dimension_semantics=("parallel",),
            vmem_limit_bytes=_VMEM_LIMIT,
        ),
    )(relu2, xs, scale, k3w, k3b)


def _node_body(apply_relu, aggr_ref, h_ref, root_ref, bias_ref, o_ref):
    h_new = aggr_ref[...] + jnp.dot(h_ref[...], root_ref[...],
                                    preferred_element_type=_F32) + bias_ref[...]
    if apply_relu:
        h_new = jnp.maximum(h_new, 0.0)
    o_ref[...] = h_new


def _node_update(aggr, h, root, bias2d, *, tile, apply_relu):
    n_pad, width = h.shape
    return pl.pallas_call(
        functools.partial(_node_body, apply_relu),
        out_shape=jax.ShapeDtypeStruct((n_pad, width), _F32),
        grid=(n_pad // tile,),
        in_specs=[_tile_spec((tile, width)),
                  _tile_spec((tile, width)),
                  _const_spec(root.shape),
                  _const_spec(bias2d.shape)],
        out_specs=_tile_spec((tile, width)),
        compiler_params=pltpu.CompilerParams(
            dimension_semantics=("parallel",),
            vmem_limit_bytes=_VMEM_LIMIT,
        ),
    )(aggr, h, root, bias2d)


@jax.jit
def _forward(fc1_w, fc1_b, k1_w, k1_b, k2_w, k2_b, k3_w, k3_b, root, bias,
             fc2_w, fc2_b, x, ea, src, tgt, scale):
    depth = 3
    edge_tile = 1024
    node_tile = 2048
    n = x.shape[0]
    k_pad = ea.shape[1]
    ker_in = k1_w.shape[0]

    k1w = jnp.pad(k1_w, ((0, k_pad - ker_in), (0, 0)))
    k1b = k1_b.reshape(1, -1)
    k2b = k2_b.reshape(1, -1)
    k3b = k3_b.reshape(1, -1)
    bias2d = bias.reshape(1, -1)

    relu2 = _edge_mlp(ea, k1w, k1b, k2_w, k2b, tile=4096)

    h = x * fc1_w[0][None, :] + fc1_b[None, :]

    for d in range(depth):
        xs = jnp.take(h, src, axis=0)
        msg = _messages(relu2, xs, scale, k3_w, k3b, tile=edge_tile)
        aggr = jax.ops.segment_sum(msg, tgt, num_segments=n)
        h = _node_update(aggr, h, root, bias2d,
                         tile=node_tile, apply_relu=(d != depth - 1))

    return h @ fc2_w + fc2_b[None, :]


def kernel(fc1_w, fc1_b, k1_w, k1_b, k2_w, k2_b, k3_w, k3_b, root, bias,
           fc2_w, fc2_b, x, ea, src, tgt, scale):
    return _forward(fc1_w, fc1_b, k1_w, k1_b, k2_w, k2_b, k3_w, k3_b, root,
                    bias, fc2_w, fc2_b, x, ea, src, tgt, scale)


# speedup vs baseline: 1.8386x; 1.8386x over previous
import functools

import jax
import jax.numpy as jnp
from jax.experimental import pallas as pl
from jax.experimental.pallas import tpu as pltpu

_F32 = jnp.float32
_VMEM_LIMIT = 96 * 1024 * 1024


def _tile_spec(block_shape):
    nd = len(block_shape)
    return pl.BlockSpec(block_shape, lambda i, _nd=nd: (i,) + (0,) * (_nd - 1))


def _const_spec(shape):
    nd = len(shape)
    return pl.BlockSpec(shape, lambda i, _nd=nd: (0,) * _nd)


def _edge_mlp_body(ea_ref, k1w_ref, k1b_ref, k2w_ref, k2b_ref, o_ref):
    e = jnp.dot(ea_ref[...], k1w_ref[...], preferred_element_type=_F32) + k1b_ref[...]
    e = jnp.maximum(e, 0.0)
    e = jnp.dot(e, k2w_ref[...], preferred_element_type=_F32) + k2b_ref[...]
    o_ref[...] = jnp.maximum(e, 0.0)


def _edge_mlp(ea, k1w, k1b, k2w, k2b, *, tile):
    e_pad, k_pad = ea.shape
    wk = k2w.shape[1]
    return pl.pallas_call(
        _edge_mlp_body,
        out_shape=jax.ShapeDtypeStruct((e_pad, wk), _F32),
        grid=(e_pad // tile,),
        in_specs=[_tile_spec((tile, k_pad)),
                  _const_spec(k1w.shape), _const_spec(k1b.shape),
                  _const_spec(k2w.shape), _const_spec(k2b.shape)],
        out_specs=_tile_spec((tile, wk)),
        compiler_params=pltpu.CompilerParams(
            dimension_semantics=("parallel",),
            vmem_limit_bytes=_VMEM_LIMIT,
        ),
    )(ea, k1w, k1b, k2w, k2b)


def _msg_body(r2_ref, xs_ref, sc_ref, k3w_ref, k3b_ref, rep_ref, o_ref):
    wflat = jnp.dot(r2_ref[...], k3w_ref[...], preferred_element_type=_F32)
    wflat = wflat + k3b_ref[...]
    xr = jnp.dot(xs_ref[...], rep_ref[...], preferred_element_type=_F32)
    prod = xr * wflat
    s = prod[:, 0:128]
    for k in range(1, 8):
        s = s + prod[:, 128 * k:128 * (k + 1)]
    msg = (s[:, 0:32] + s[:, 32:64]) + (s[:, 64:96] + s[:, 96:128])
    o_ref[...] = msg * sc_ref[...]


def _messages(relu2, xs, scale, k3w, k3b, rep, *, tile):
    e_pad = relu2.shape[0]
    width = xs.shape[1]
    return pl.pallas_call(
        _msg_body,
        out_shape=jax.ShapeDtypeStruct((e_pad, width), _F32),
        grid=(e_pad // tile,),
        in_specs=[_tile_spec((tile, relu2.shape[1])),
                  _tile_spec((tile, width)),
                  _tile_spec((tile, 1)),
                  _const_spec(k3w.shape), _const_spec(k3b.shape),
                  _const_spec(rep.shape)],
        out_specs=_tile_spec((tile, width)),
        compiler_params=pltpu.CompilerParams(
            dimension_semantics=("parallel",),
            vmem_limit_bytes=_VMEM_LIMIT,
        ),
    )(relu2, xs, scale, k3w, k3b, rep)


def _node_body(apply_relu, aggr_ref, h_ref, root_ref, bias_ref, o_ref):
    h_new = aggr_ref[...] + jnp.dot(h_ref[...], root_ref[...],
                                    preferred_element_type=_F32) + bias_ref[...]
    if apply_relu:
        h_new = jnp.maximum(h_new, 0.0)
    o_ref[...] = h_new


def _node_update(aggr, h, root, bias2d, *, tile, apply_relu):
    n_pad, width = h.shape
    return pl.pallas_call(
        functools.partial(_node_body, apply_relu),
        out_shape=jax.ShapeDtypeStruct((n_pad, width), _F32),
        grid=(n_pad // tile,),
        in_specs=[_tile_spec((tile, width)),
                  _tile_spec((tile, width)),
                  _const_spec(root.shape),
                  _const_spec(bias2d.shape)],
        out_specs=_tile_spec((tile, width)),
        compiler_params=pltpu.CompilerParams(
            dimension_semantics=("parallel",),
            vmem_limit_bytes=_VMEM_LIMIT,
        ),
    )(aggr, h, root, bias2d)


@jax.jit
def _forward(fc1_w, fc1_b, k1_w, k1_b, k2_w, k2_b, k3_w, k3_b, root, bias,
             fc2_w, fc2_b, x, ea, src, tgt, scale):
    depth = 3
    edge_tile = 1024
    node_tile = 2048
    n = x.shape[0]
    k_pad = ea.shape[1]
    ker_in = k1_w.shape[0]

    k1w = jnp.pad(k1_w, ((0, k_pad - ker_in), (0, 0)))
    k1b = k1_b.reshape(1, -1)
    k2b = k2_b.reshape(1, -1)
    k3b = k3_b.reshape(1, -1)
    bias2d = bias.reshape(1, -1)

    relu2 = _edge_mlp(ea, k1w, k1b, k2_w, k2b, tile=4096)

    width = root.shape[0]
    j = jnp.arange(width * width)
    rep = (jnp.arange(width)[:, None] == (j // width)[None, :]).astype(_F32)

    h = x * fc1_w[0][None, :] + fc1_b[None, :]

    for d in range(depth):
        xs = jnp.take(h, src, axis=0)
        msg = _messages(relu2, xs, scale, k3_w, k3b, rep, tile=edge_tile)
        aggr = jax.ops.segment_sum(msg, tgt, num_segments=n)
        h = _node_update(aggr, h, root, bias2d,
                         tile=node_tile, apply_relu=(d != depth - 1))

    return h @ fc2_w + fc2_b[None, :]


def kernel(fc1_w, fc1_b, k1_w, k1_b, k2_w, k2_b, k3_w, k3_b, root, bias,
           fc2_w, fc2_b, x, ea, src, tgt, scale):
    return _forward(fc1_w, fc1_b, k1_w, k1_b, k2_w, k2_b, k3_w, k3_b, root,
                    bias, fc2_w, fc2_b, x, ea, src, tgt, scale)


# speedup vs baseline: 3.0062x; 1.6350x over previous
import functools

import jax
import jax.numpy as jnp
from jax.experimental import pallas as pl
from jax.experimental.pallas import tpu as pltpu

_F32 = jnp.float32
_VMEM_LIMIT = 96 * 1024 * 1024


def _tile_spec(block_shape):
    nd = len(block_shape)
    return pl.BlockSpec(block_shape, lambda i, _nd=nd: (i,) + (0,) * (_nd - 1))


def _const_spec(shape):
    nd = len(shape)
    return pl.BlockSpec(shape, lambda i, _nd=nd: (0,) * _nd)


def _edge_mlp_body(ea_ref, k1w_ref, k1b_ref, k2w_ref, k2b_ref, o_ref):
    e = jnp.dot(ea_ref[...], k1w_ref[...], preferred_element_type=_F32) + k1b_ref[...]
    e = jnp.maximum(e, 0.0)
    e = jnp.dot(e, k2w_ref[...], preferred_element_type=_F32) + k2b_ref[...]
    o_ref[...] = jnp.maximum(e, 0.0)


def _edge_mlp(ea, k1w, k1b, k2w, k2b, *, tile):
    e_pad, k_pad = ea.shape
    wk = k2w.shape[1]
    return pl.pallas_call(
        _edge_mlp_body,
        out_shape=jax.ShapeDtypeStruct((e_pad, wk), _F32),
        grid=(e_pad // tile,),
        in_specs=[_tile_spec((tile, k_pad)),
                  _const_spec(k1w.shape), _const_spec(k1b.shape),
                  _const_spec(k2w.shape), _const_spec(k2b.shape)],
        out_specs=_tile_spec((tile, wk)),
        compiler_params=pltpu.CompilerParams(
            dimension_semantics=("parallel",),
            vmem_limit_bytes=_VMEM_LIMIT,
        ),
    )(ea, k1w, k1b, k2w, k2b)


_GATHER_UNROLL = 16


def _msg_body(r2_ref, src_ref, sc_ref, h_ref, k3w_ref, k3b_ref, rep_ref,
              o_ref, xs_scr):
    tile = o_ref.shape[0]

    def gather_chunk(ci, carry):
        base = ci * _GATHER_UNROLL
        for u in range(_GATHER_UNROLL):
            idx = src_ref[0, 0, base + u]
            xs_scr[pl.ds(base + u, 1), :] = h_ref[pl.ds(idx, 1), :]
        return carry

    jax.lax.fori_loop(0, tile // _GATHER_UNROLL, gather_chunk, 0)

    wflat = jnp.dot(r2_ref[...], k3w_ref[...], preferred_element_type=_F32)
    wflat = wflat + k3b_ref[...]
    xr = jnp.dot(xs_scr[...], rep_ref[...], preferred_element_type=_F32)
    prod = xr * wflat
    s = prod[:, 0:128]
    for k in range(1, 8):
        s = s + prod[:, 128 * k:128 * (k + 1)]
    msg = (s[:, 0:32] + s[:, 32:64]) + (s[:, 64:96] + s[:, 96:128])
    o_ref[...] = msg * sc_ref[...]


def _messages(relu2, src2d, scale, h, k3w, k3b, rep, *, tile):
    e_pad = relu2.shape[0]
    width = h.shape[1]
    return pl.pallas_call(
        _msg_body,
        out_shape=jax.ShapeDtypeStruct((e_pad, width), _F32),
        grid=(e_pad // tile,),
        in_specs=[_tile_spec((tile, relu2.shape[1])),
                  pl.BlockSpec((1, 1, tile), lambda i: (i, 0, 0),
                               memory_space=pltpu.SMEM),
                  _tile_spec((tile, 1)),
                  _const_spec(h.shape),
                  _const_spec(k3w.shape), _const_spec(k3b.shape),
                  _const_spec(rep.shape)],
        out_specs=_tile_spec((tile, width)),
        scratch_shapes=[pltpu.VMEM((tile, width), _F32)],
        compiler_params=pltpu.CompilerParams(
            dimension_semantics=("parallel",),
            vmem_limit_bytes=_VMEM_LIMIT,
        ),
    )(relu2, src2d, scale, h, k3w, k3b, rep)


def _node_body(apply_relu, aggr_ref, h_ref, root_ref, bias_ref, o_ref):
    h_new = aggr_ref[...] + jnp.dot(h_ref[...], root_ref[...],
                                    preferred_element_type=_F32) + bias_ref[...]
    if apply_relu:
        h_new = jnp.maximum(h_new, 0.0)
    o_ref[...] = h_new


def _node_update(aggr, h, root, bias2d, *, tile, apply_relu):
    n_pad, width = h.shape
    return pl.pallas_call(
        functools.partial(_node_body, apply_relu),
        out_shape=jax.ShapeDtypeStruct((n_pad, width), _F32),
        grid=(n_pad // tile,),
        in_specs=[_tile_spec((tile, width)),
                  _tile_spec((tile, width)),
                  _const_spec(root.shape),
                  _const_spec(bias2d.shape)],
        out_specs=_tile_spec((tile, width)),
        compiler_params=pltpu.CompilerParams(
            dimension_semantics=("parallel",),
            vmem_limit_bytes=_VMEM_LIMIT,
        ),
    )(aggr, h, root, bias2d)


@jax.jit
def _forward(fc1_w, fc1_b, k1_w, k1_b, k2_w, k2_b, k3_w, k3_b, root, bias,
             fc2_w, fc2_b, x, ea, src, tgt, scale):
    depth = 3
    edge_tile = 1024
    node_tile = 2048
    n = x.shape[0]
    k_pad = ea.shape[1]
    ker_in = k1_w.shape[0]

    k1w = jnp.pad(k1_w, ((0, k_pad - ker_in), (0, 0)))
    k1b = k1_b.reshape(1, -1)
    k2b = k2_b.reshape(1, -1)
    k3b = k3_b.reshape(1, -1)
    bias2d = bias.reshape(1, -1)

    relu2 = _edge_mlp(ea, k1w, k1b, k2_w, k2b, tile=4096)

    width = root.shape[0]
    j = jnp.arange(width * width)
    rep = (jnp.arange(width)[:, None] == (j // width)[None, :]).astype(_F32)

    h = x * fc1_w[0][None, :] + fc1_b[None, :]

    src2d = src.reshape(-1, 1, edge_tile)

    for d in range(depth):
        msg = _messages(relu2, src2d, scale, h, k3_w, k3b, rep, tile=edge_tile)
        aggr = jax.ops.segment_sum(msg, tgt, num_segments=n)
        h = _node_update(aggr, h, root, bias2d,
                         tile=node_tile, apply_relu=(d != depth - 1))

    return h @ fc2_w + fc2_b[None, :]


def kernel(fc1_w, fc1_b, k1_w, k1_b, k2_w, k2_b, k3_w, k3_b, root, bias,
           fc2_w, fc2_b, x, ea, src, tgt, scale):
    return _forward(fc1_w, fc1_b, k1_w, k1_b, k2_w, k2_b, k3_w, k3_b, root,
                    bias, fc2_w, fc2_b, x, ea, src, tgt, scale)
